# baseline (speedup 1.0000x reference)
import functools

import jax
import jax.numpy as jnp
from jax import lax
from jax.experimental import pallas as pl
from jax.experimental.pallas import tpu as pltpu
from jax.experimental.pallas import tpu_sc as plsc

N = 10000
E = 320000
D = 128
H = 128

NC = 2
NS = 16
NW = NC * NS
LANES = 16

RB = 1000
NB = N // RB

E_PER_TILE32 = E // NW
E_PER_TILE16 = E // NS
ROWS_PER_TILE = N // NS
K1 = 125
NCHUNK1 = E_PER_TILE16 // K1
NBUF = 5

_mesh = plsc.VectorSubcoreMesh(core_axis_name="c", subcore_axis_name="s")


def _deg_body(dst_hbm, zeros_hbm, out_hbm, idx_v, deg_v):
    c = lax.axis_index("c")
    s = lax.axis_index("s")
    wid = s * NC + c
    base = wid * E_PER_TILE32
    pltpu.sync_copy(zeros_hbm, deg_v)
    pltpu.sync_copy(dst_hbm.at[pl.ds(base, E_PER_TILE32)], idx_v)
    ones = jnp.full((LANES,), 1.0, dtype=jnp.float32)

    def body(i, _):
        idx = idx_v[pl.ds(i * LANES, LANES)]
        plsc.addupdate_scatter(deg_v, [idx], ones)
        return 0

    lax.fori_loop(0, E_PER_TILE32 // LANES, body, 0)
    pltpu.sync_copy(deg_v, out_hbm.at[wid])


@functools.partial(
    pl.kernel,
    out_type=jax.ShapeDtypeStruct((NW, N), jnp.float32),
    mesh=_mesh,
    scratch_types=[
        pltpu.VMEM((E_PER_TILE32,), jnp.int32),
        pltpu.VMEM((N,), jnp.float32),
    ],
    compiler_params=pltpu.CompilerParams(needs_layout_passes=False, use_tc_tiling_on_sc=False),
)
def _deg_kernel(dst_hbm, zeros_hbm, out_hbm, idx_v, deg_v):
    _deg_body(dst_hbm, zeros_hbm, out_hbm, idx_v, deg_v)


def _agg_body(hs_hbm, srcoff_hbm, dst_hbm, out_hbm, sidx_v, didx_v, rows_v,
              agg_s, sem_g, sem_s):
    c = lax.axis_index("c")
    s = lax.axis_index("s")
    pltpu.sync_copy(srcoff_hbm.at[c, s], sidx_v)
    pltpu.sync_copy(dst_hbm.at[s], didx_v)
    r0 = s * ROWS_PER_TILE
    pltpu.sync_copy(hs_hbm.at[pl.ds(c * N + r0, ROWS_PER_TILE)],
                    agg_s.at[pl.ds(r0, ROWS_PER_TILE)])
    plsc.subcore_barrier()

    for b in range(NBUF):
        pltpu.async_copy(hs_hbm.at[sidx_v.at[b]], rows_v.at[b], sem_g[b])

    def body(i, _):
        jbase = i * NBUF
        for b in range(NBUF):
            j = jbase + b
            pltpu.make_async_copy(hs_hbm.at[sidx_v.at[j]], rows_v.at[b],
                                  sem_g[b]).wait()
            pltpu.async_copy(rows_v.at[b], agg_s.at[didx_v.at[j]], sem_s[b],
                             add=True)
        for b in range(NBUF):
            j = jbase + b
            jn = j + NBUF

            @pl.when(jn < NCHUNK1)
            def _():
                pltpu.make_async_copy(rows_v.at[b], agg_s.at[didx_v.at[j]],
                                      sem_s[b]).wait()
                pltpu.async_copy(hs_hbm.at[sidx_v.at[jn]], rows_v.at[b],
                                 sem_g[b])

        return 0

    lax.fori_loop(0, NCHUNK1 // NBUF, body, 0)
    for b in range(NBUF):
        pltpu.make_async_copy(rows_v.at[b], agg_s.at[didx_v.at[0]],
                              sem_s[b]).wait()
    plsc.subcore_barrier()
    pltpu.sync_copy(agg_s.at[pl.ds(r0, ROWS_PER_TILE)],
                    out_hbm.at[pl.ds(c * N + r0, ROWS_PER_TILE)])


@functools.partial(
    pl.kernel,
    out_type=jax.ShapeDtypeStruct((2 * N, 64), jnp.float32),
    mesh=_mesh,
    scratch_types=[
        pltpu.VMEM((NCHUNK1, K1), jnp.int32),
        pltpu.VMEM((NCHUNK1, K1), jnp.int32),
        pltpu.VMEM((NBUF, K1, 64), jnp.float32),
        pltpu.VMEM_SHARED((N, 64), jnp.float32),
        [pltpu.SemaphoreType.DMA] * NBUF,
        [pltpu.SemaphoreType.DMA] * NBUF,
    ],
    compiler_params=pltpu.CompilerParams(needs_layout_passes=False, use_tc_tiling_on_sc=False),
)
def _agg_kernel(hs_hbm, srcoff_hbm, dst_hbm, out_hbm, sidx_v, didx_v, rows_v,
                agg_s, sem_g, sem_s):
    _agg_body(hs_hbm, srcoff_hbm, dst_hbm, out_hbm, sidx_v, didx_v, rows_v,
              agg_s, sem_g, sem_s)


NPAD = 10240
STRIP = NPAD // NW


def _aggz_body(zs_hbm, src_hbm, dst_hbm, zeros_hbm, dinv_hbm, cb_hbm, out_hbm,
               zs_v, sidx_v, didx_v, acc_v, blk_v, dinv_v, cb_v, out_v,
               stage_s):
    c = lax.axis_index("c")
    s = lax.axis_index("s")
    ebase = s * E_PER_TILE16
    pltpu.sync_copy(zeros_hbm, acc_v)
    pltpu.sync_copy(zs_hbm, zs_v)
    pltpu.sync_copy(src_hbm.at[pl.ds(ebase, E_PER_TILE16)], sidx_v)
    pltpu.sync_copy(dst_hbm.at[pl.ds(ebase, E_PER_TILE16)], didx_v)

    def body(i, _):
        sl = pl.ds(i * LANES, LANES)
        vals = plsc.load_gather(zs_v, [sidx_v[sl]])
        plsc.addupdate_scatter(acc_v, [didx_v[sl]], vals)
        return 0

    lax.fori_loop(0, E_PER_TILE16 // LANES, body, 0)
    pltpu.sync_copy(acc_v, stage_s.at[s])
    plsc.subcore_barrier()

    r0 = c * (NPAD // 2) + s * STRIP
    pltpu.sync_copy(stage_s.at[:, pl.ds(r0, STRIP)], blk_v)
    pltpu.sync_copy(dinv_hbm.at[pl.ds(r0, STRIP)], dinv_v)
    pltpu.sync_copy(cb_hbm.at[pl.ds(0, LANES)], cb_v)
    cb = cb_v[...]
    for g in range(STRIP // LANES):
        gl = pl.ds(g * LANES, LANES)
        tot = zs_v[pl.ds(r0 + g * LANES, LANES)]
        for t in range(NS):
            tot = tot + blk_v[t, gl]
        out_v[gl] = tot * dinv_v[gl] + cb
    pltpu.sync_copy(out_v, out_hbm.at[pl.ds(r0, STRIP)])


@functools.partial(
    pl.kernel,
    out_type=jax.ShapeDtypeStruct((NPAD,), jnp.float32),
    mesh=_mesh,
    scratch_types=[
        pltpu.VMEM((NPAD,), jnp.float32),
        pltpu.VMEM((E_PER_TILE16,), jnp.int32),
        pltpu.VMEM((E_PER_TILE16,), jnp.int32),
        pltpu.VMEM((NPAD,), jnp.float32),
        pltpu.VMEM((NS, STRIP), jnp.float32),
        pltpu.VMEM((STRIP,), jnp.float32),
        pltpu.VMEM((LANES,), jnp.float32),
        pltpu.VMEM((STRIP,), jnp.float32),
        pltpu.VMEM_SHARED((NS, NPAD), jnp.float32),
    ],
    compiler_params=pltpu.CompilerParams(needs_layout_passes=False, use_tc_tiling_on_sc=False),
)
def _aggz_kernel(zs_hbm, src_hbm, dst_hbm, zeros_hbm, dinv_hbm, cb_hbm,
                 out_hbm, zs_v, sidx_v, didx_v, acc_v, blk_v, dinv_v, cb_v,
                 out_v, stage_s):
    _aggz_body(zs_hbm, src_hbm, dst_hbm, zeros_hbm, dinv_hbm, cb_hbm, out_hbm,
               zs_v, sidx_v, didx_v, acc_v, blk_v, dinv_v, cb_v, out_v,
               stage_s)


def _tc1_body(x_ref, w1_ref, part_ref, hs_ref, dinv_ref):
    deg = jnp.sum(part_ref[0], axis=1)[:, None] + 1.0
    dinv = lax.rsqrt(deg)
    y = jnp.dot(x_ref[0], w1_ref[0], preferred_element_type=jnp.float32)
    hs_ref[...] = y * dinv
    dinv_ref[0] = dinv


def _tc1(x3, w1, parts3):
    return pl.pallas_call(
        _tc1_body,
        grid=(NB, 2),
        in_specs=[
            pl.BlockSpec((1, RB, D), lambda i, h: (i, 0, 0)),
            pl.BlockSpec((1, D, 64), lambda i, h: (h, 0, 0)),
            pl.BlockSpec((1, RB, NW), lambda i, h: (i, 0, 0)),
        ],
        out_specs=[
            pl.BlockSpec((RB, 64), lambda i, h: (h * NB + i, 0)),
            pl.BlockSpec((1, RB, 1), lambda i, h: (i, 0, 0)),
        ],
        out_shape=[
            jax.ShapeDtypeStruct((2 * N, 64), jnp.float32),
            jax.ShapeDtypeStruct((NB, RB, 1), jnp.float32),
        ],
    )(x3, w1, parts3)


def _tc2_body(a0_ref, a1_ref, dinv_ref, b1_ref, w2_ref, wo_ref, b2_ref,
              bout_ref, zs_ref, cb_ref):
    dinv = dinv_ref[0]
    b1 = b1_ref[...]
    h0 = jnp.maximum(a0_ref[...] * dinv + b1[None, :64], 0.0)
    h1 = jnp.maximum(a1_ref[...] * dinv + b1[None, 64:], 0.0)
    w2o = jnp.dot(w2_ref[...], wo_ref[...],
                  preferred_element_type=jnp.float32)
    z = (jnp.dot(h0, w2o[:64], preferred_element_type=jnp.float32)
         + jnp.dot(h1, w2o[64:], preferred_element_type=jnp.float32))
    zs_ref[0] = z * dinv
    cb = jnp.dot(b2_ref[...][None, :], w2o,
                 preferred_element_type=jnp.float32)[0, 0] + bout_ref[0]
    cb_ref[...] = jnp.full((H,), cb, dtype=jnp.float32)


def _tc2(agg, dinv2, b1, w2, wout, b2, bout):
    return pl.pallas_call(
        _tc2_body,
        grid=(NB,),
        in_specs=[
            pl.BlockSpec((RB, 64), lambda i: (i, 0)),
            pl.BlockSpec((RB, 64), lambda i: (NB + i, 0)),
            pl.BlockSpec((1, RB, 1), lambda i: (i, 0, 0)),
            pl.BlockSpec((H,), lambda i: (0,)),
            pl.BlockSpec((H, H), lambda i: (0, 0)),
            pl.BlockSpec((H, 1), lambda i: (0, 0)),
            pl.BlockSpec((H,), lambda i: (0,)),
            pl.BlockSpec((1,), lambda i: (0,)),
        ],
        out_specs=[
            pl.BlockSpec((1, RB, 1), lambda i: (i, 0, 0)),
            pl.BlockSpec((H,), lambda i: (0,)),
        ],
        out_shape=[
            jax.ShapeDtypeStruct((NB, RB, 1), jnp.float32),
            jax.ShapeDtypeStruct((H,), jnp.float32),
        ],
    )(agg, agg, dinv2, b1, w2, wout, b2, bout)


@jax.jit
def kernel(x, edge_index, W1, b1, W2, b2, Wout, bout):
    src = edge_index[0].astype(jnp.int32)
    dst = edge_index[1].astype(jnp.int32)
    srcoff = jnp.stack([src, src + N])
    zeros_n = jnp.zeros((N,), jnp.float32)

    w1s = jnp.transpose(W1.reshape(D, 2, 64), (1, 0, 2))
    deg_parts = _deg_kernel(dst, zeros_n)
    hs, dinv2 = _tc1(x.reshape(NB, RB, D), w1s,
                     deg_parts.T.reshape(NB, RB, NW))
    agg = _agg_kernel(hs, srcoff.reshape(2, NS, NCHUNK1, K1),
                      dst.reshape(NS, NCHUNK1, K1))
    zs2, cb_arr = _tc2(agg, dinv2, b1, W2, Wout, b2, bout)
    pad = jnp.zeros((NPAD - N,), jnp.float32)
    zs_pad = jnp.concatenate([zs2.reshape(N), pad])
    dinv_pad = jnp.concatenate([dinv2.reshape(N), pad])
    zeros_p = jnp.zeros((NPAD,), jnp.float32)
    out_pad = _aggz_kernel(zs_pad, src, dst, zeros_p, dinv_pad, cb_arr)
    return out_pad[:N]


# speedup vs baseline: 1.0296x; 1.0296x over previous
import functools

import jax
import jax.numpy as jnp
from jax import lax
from jax.experimental import pallas as pl
from jax.experimental.pallas import tpu as pltpu
from jax.experimental.pallas import tpu_sc as plsc

N = 10000
E = 320000
D = 128
H = 128

NC = 2
NS = 16
NW = NC * NS
LANES = 16

RB = 1000
NB = N // RB

E_PER_TILE32 = E // NW
E_PER_TILE16 = E // NS
ROWS_PER_TILE = N // NS
K1 = 125
NCHUNK1 = E_PER_TILE16 // K1
NBUF = 5

_mesh = plsc.VectorSubcoreMesh(core_axis_name="c", subcore_axis_name="s")


def _deg_body(dst_hbm, zeros_hbm, out_hbm, idx_v, deg_v):
    c = lax.axis_index("c")
    s = lax.axis_index("s")
    wid = s * NC + c
    base = wid * E_PER_TILE32
    pltpu.sync_copy(zeros_hbm, deg_v)
    pltpu.sync_copy(dst_hbm.at[pl.ds(base, E_PER_TILE32)], idx_v)
    ones = jnp.full((LANES,), 1.0, dtype=jnp.float32)

    def body(i, _):
        idx = idx_v[pl.ds(i * LANES, LANES)]
        plsc.addupdate_scatter(deg_v, [idx], ones)
        return 0

    lax.fori_loop(0, E_PER_TILE32 // LANES, body, 0)
    pltpu.sync_copy(deg_v, out_hbm.at[wid])


@functools.partial(
    pl.kernel,
    out_type=jax.ShapeDtypeStruct((NW, N), jnp.float32),
    mesh=_mesh,
    scratch_types=[
        pltpu.VMEM((E_PER_TILE32,), jnp.int32),
        pltpu.VMEM((N,), jnp.float32),
    ],
    compiler_params=pltpu.CompilerParams(needs_layout_passes=False, use_tc_tiling_on_sc=False),
)
def _deg_kernel(dst_hbm, zeros_hbm, out_hbm, idx_v, deg_v):
    _deg_body(dst_hbm, zeros_hbm, out_hbm, idx_v, deg_v)


def _agg_body(hs_hbm, srcoff_hbm, dst_hbm, out_hbm, sidx_v, didx_v, rows_v,
              agg_s, sem_g, sem_s):
    c = lax.axis_index("c")
    s = lax.axis_index("s")
    pltpu.sync_copy(srcoff_hbm.at[c, s], sidx_v)
    pltpu.sync_copy(dst_hbm.at[s], didx_v)
    r0 = s * ROWS_PER_TILE
    pltpu.sync_copy(hs_hbm.at[pl.ds(c * N + r0, ROWS_PER_TILE)],
                    agg_s.at[pl.ds(r0, ROWS_PER_TILE)])
    plsc.subcore_barrier()

    for b in range(NBUF):
        pltpu.async_copy(hs_hbm.at[sidx_v.at[b]], rows_v.at[b], sem_g[b])

    def body(i, _):
        jbase = i * NBUF
        for b in range(NBUF):
            j = jbase + b
            pltpu.make_async_copy(hs_hbm.at[sidx_v.at[j]], rows_v.at[b],
                                  sem_g[b]).wait()
            pltpu.async_copy(rows_v.at[b], agg_s.at[didx_v.at[j]], sem_s[b],
                             add=True)
        for b in range(NBUF):
            j = jbase + b
            jn = j + NBUF

            @pl.when(jn < NCHUNK1)
            def _():
                pltpu.make_async_copy(rows_v.at[b], agg_s.at[didx_v.at[j]],
                                      sem_s[b]).wait()
                pltpu.async_copy(hs_hbm.at[sidx_v.at[jn]], rows_v.at[b],
                                 sem_g[b])

        return 0

    lax.fori_loop(0, NCHUNK1 // NBUF, body, 0)
    for b in range(NBUF):
        pltpu.make_async_copy(rows_v.at[b], agg_s.at[didx_v.at[0]],
                              sem_s[b]).wait()
    plsc.subcore_barrier()
    pltpu.sync_copy(agg_s.at[pl.ds(r0, ROWS_PER_TILE)],
                    out_hbm.at[pl.ds(c * N + r0, ROWS_PER_TILE)])


@functools.partial(
    pl.kernel,
    out_type=jax.ShapeDtypeStruct((2 * N, 64), jnp.float32),
    mesh=_mesh,
    scratch_types=[
        pltpu.VMEM((NCHUNK1, K1), jnp.int32),
        pltpu.VMEM((NCHUNK1, K1), jnp.int32),
        pltpu.VMEM((NBUF, K1, 64), jnp.float32),
        pltpu.VMEM_SHARED((N, 64), jnp.float32),
        [pltpu.SemaphoreType.DMA] * NBUF,
        [pltpu.SemaphoreType.DMA] * NBUF,
    ],
    compiler_params=pltpu.CompilerParams(needs_layout_passes=False, use_tc_tiling_on_sc=False),
)
def _agg_kernel(hs_hbm, srcoff_hbm, dst_hbm, out_hbm, sidx_v, didx_v, rows_v,
                agg_s, sem_g, sem_s):
    _agg_body(hs_hbm, srcoff_hbm, dst_hbm, out_hbm, sidx_v, didx_v, rows_v,
              agg_s, sem_g, sem_s)


NPAD = 10240
STRIP = NPAD // NW


def _aggz_body(zs_hbm, src_hbm, dst_hbm, zeros_hbm, dinv_hbm, cb_hbm, out_hbm,
               zs_v, sidx_v, didx_v, acc_v, blk_v, dinv_v, cb_v, out_v,
               stage_s):
    c = lax.axis_index("c")
    s = lax.axis_index("s")
    ebase = s * E_PER_TILE16
    pltpu.sync_copy(zeros_hbm, acc_v)
    pltpu.sync_copy(zs_hbm, zs_v)
    pltpu.sync_copy(src_hbm.at[pl.ds(ebase, E_PER_TILE16)], sidx_v)
    pltpu.sync_copy(dst_hbm.at[pl.ds(ebase, E_PER_TILE16)], didx_v)

    def body(i, _):
        sl = pl.ds(i * LANES, LANES)
        vals = plsc.load_gather(zs_v, [sidx_v[sl]])
        plsc.addupdate_scatter(acc_v, [didx_v[sl]], vals)
        return 0

    lax.fori_loop(0, E_PER_TILE16 // LANES, body, 0)
    pltpu.sync_copy(acc_v, stage_s.at[s])
    plsc.subcore_barrier()

    r0 = c * (NPAD // 2) + s * STRIP
    pltpu.sync_copy(stage_s.at[:, pl.ds(r0, STRIP)], blk_v)
    pltpu.sync_copy(dinv_hbm.at[pl.ds(r0, STRIP)], dinv_v)
    pltpu.sync_copy(cb_hbm.at[pl.ds(0, LANES)], cb_v)
    cb = cb_v[...]
    for g in range(STRIP // LANES):
        gl = pl.ds(g * LANES, LANES)
        tot = zs_v[pl.ds(r0 + g * LANES, LANES)]
        for t in range(NS):
            tot = tot + blk_v[t, gl]
        out_v[gl] = tot * dinv_v[gl] + cb
    pltpu.sync_copy(out_v, out_hbm.at[pl.ds(r0, STRIP)])


@functools.partial(
    pl.kernel,
    out_type=jax.ShapeDtypeStruct((NPAD,), jnp.float32),
    mesh=_mesh,
    scratch_types=[
        pltpu.VMEM((NPAD,), jnp.float32),
        pltpu.VMEM((E_PER_TILE16,), jnp.int32),
        pltpu.VMEM((E_PER_TILE16,), jnp.int32),
        pltpu.VMEM((NPAD,), jnp.float32),
        pltpu.VMEM((NS, STRIP), jnp.float32),
        pltpu.VMEM((STRIP,), jnp.float32),
        pltpu.VMEM((LANES,), jnp.float32),
        pltpu.VMEM((STRIP,), jnp.float32),
        pltpu.VMEM_SHARED((NS, NPAD), jnp.float32),
    ],
    compiler_params=pltpu.CompilerParams(needs_layout_passes=False, use_tc_tiling_on_sc=False),
)
def _aggz_kernel(zs_hbm, src_hbm, dst_hbm, zeros_hbm, dinv_hbm, cb_hbm,
                 out_hbm, zs_v, sidx_v, didx_v, acc_v, blk_v, dinv_v, cb_v,
                 out_v, stage_s):
    _aggz_body(zs_hbm, src_hbm, dst_hbm, zeros_hbm, dinv_hbm, cb_hbm, out_hbm,
               zs_v, sidx_v, didx_v, acc_v, blk_v, dinv_v, cb_v, out_v,
               stage_s)


def _tc1_body(x_ref, w1_ref, part_ref, hs_ref, dinv_ref):
    deg = jnp.sum(part_ref[0], axis=0) + 1.0
    dinv = lax.rsqrt(deg)
    y = jnp.dot(x_ref[0], w1_ref[0], preferred_element_type=jnp.float32)
    hs_ref[...] = y * dinv[:, None]
    dinv_ref[0, 0, :] = dinv


def _tc1(x3, w1, parts3):
    return pl.pallas_call(
        _tc1_body,
        grid=(NB, 2),
        in_specs=[
            pl.BlockSpec((1, RB, D), lambda i, h: (i, 0, 0)),
            pl.BlockSpec((1, D, 64), lambda i, h: (h, 0, 0)),
            pl.BlockSpec((1, NW, RB), lambda i, h: (i, 0, 0)),
        ],
        out_specs=[
            pl.BlockSpec((RB, 64), lambda i, h: (h * NB + i, 0)),
            pl.BlockSpec((1, 1, RB), lambda i, h: (i, 0, 0)),
        ],
        out_shape=[
            jax.ShapeDtypeStruct((2 * N, 64), jnp.float32),
            jax.ShapeDtypeStruct((NB, 1, RB), jnp.float32),
        ],
    )(x3, w1, parts3)


def _tc2_body(a0_ref, a1_ref, dinv_ref, b1_ref, w2_ref, wo_ref, b2_ref,
              bout_ref, zs_ref, cb_ref):
    dinv = dinv_ref[0, 0, :]
    b1 = b1_ref[...]
    h0 = jnp.maximum(a0_ref[...] * dinv[:, None] + b1[None, :64], 0.0)
    h1 = jnp.maximum(a1_ref[...] * dinv[:, None] + b1[None, 64:], 0.0)
    wo = wo_ref[...][:, 0]
    w2o = jnp.sum(w2_ref[...] * wo[None, :], axis=1)
    z = (jnp.sum(h0 * w2o[None, :64], axis=1)
         + jnp.sum(h1 * w2o[None, 64:], axis=1))
    zs_ref[0, 0, :] = z * dinv
    cb = jnp.sum(b2_ref[...] * wo) + bout_ref[0]
    cb_ref[...] = jnp.full((H,), cb, dtype=jnp.float32)


def _tc2(agg, dinv2, b1, w2, wout, b2, bout):
    return pl.pallas_call(
        _tc2_body,
        grid=(NB,),
        in_specs=[
            pl.BlockSpec((RB, 64), lambda i: (i, 0)),
            pl.BlockSpec((RB, 64), lambda i: (NB + i, 0)),
            pl.BlockSpec((1, 1, RB), lambda i: (i, 0, 0)),
            pl.BlockSpec((H,), lambda i: (0,)),
            pl.BlockSpec((H, H), lambda i: (0, 0)),
            pl.BlockSpec((H, 1), lambda i: (0, 0)),
            pl.BlockSpec((H,), lambda i: (0,)),
            pl.BlockSpec((1,), lambda i: (0,)),
        ],
        out_specs=[
            pl.BlockSpec((1, 1, RB), lambda i: (i, 0, 0)),
            pl.BlockSpec((H,), lambda i: (0,)),
        ],
        out_shape=[
            jax.ShapeDtypeStruct((NB, 1, RB), jnp.float32),
            jax.ShapeDtypeStruct((H,), jnp.float32),
        ],
    )(agg, agg, dinv2, b1, w2, wout, b2, bout)


@jax.jit
def kernel(x, edge_index, W1, b1, W2, b2, Wout, bout):
    src = edge_index[0].astype(jnp.int32)
    dst = edge_index[1].astype(jnp.int32)
    srcoff = jnp.stack([src, src + N])
    zeros_n = jnp.zeros((N,), jnp.float32)

    w1s = jnp.transpose(W1.reshape(D, 2, 64), (1, 0, 2))
    deg_parts = _deg_kernel(dst, zeros_n)
    hs, dinv2 = _tc1(x.reshape(NB, RB, D), w1s,
                     deg_parts.reshape(NW, NB, RB).transpose(1, 0, 2))
    agg = _agg_kernel(hs, srcoff.reshape(2, NS, NCHUNK1, K1),
                      dst.reshape(NS, NCHUNK1, K1))
    zs2, cb_arr = _tc2(agg, dinv2, b1, W2, Wout, b2, bout)
    pad = jnp.zeros((NPAD - N,), jnp.float32)
    zs_pad = jnp.concatenate([zs2.reshape(N), pad])
    dinv_pad = jnp.concatenate([dinv2.reshape(N), pad])
    zeros_p = jnp.zeros((NPAD,), jnp.float32)
    out_pad = _aggz_kernel(zs_pad, src, dst, zeros_p, dinv_pad, cb_arr)
    return out_pad[:N]
